# baseline (speedup 1.0000x reference)
import functools

import jax
import jax.numpy as jnp
import numpy as np
from jax import lax
from jax.experimental import pallas as pl
from jax.experimental.pallas import tpu as pltpu
from jax.experimental.pallas import tpu_sc as plsc

N_POINTS = 50000
SUM_NPOINT = 320000
C = 128
N_CLUSTERS = 10000

NW = 32
SEG_W = 320
OUT_PAD = NW * SEG_W
B = 128
CW = C // 2
NCHUNK = C // 16
NEG = float("-inf")
PAD_PTS = 640

_COL_POS = np.array([32 * (c // 32) + (c % 2) * 16 + (c % 32) // 2
                     for c in range(C)])

_mesh = plsc.VectorSubcoreMesh(core_axis_name="c", subcore_axis_name="s")


def _make_seg_kernel(is_max):
    @functools.partial(
        pl.kernel,
        mesh=_mesh,
        out_type=jax.ShapeDtypeStruct((OUT_PAD * C,), jnp.float32),
        compiler_params=pltpu.CompilerParams(
            needs_layout_passes=False, use_tc_tiling_on_sc=False),
        scratch_types=[
            pltpu.VMEM(((SEG_W + 1) * C,), jnp.float32),
            pltpu.VMEM((4 * B, CW), jnp.int32),
            pltpu.VMEM((4 * B,), jnp.int32),
            pltpu.VMEM((4, B), jnp.int32),
            pltpu.VMEM((48,), jnp.int32),
            pltpu.SemaphoreType.DMA,
            pltpu.SemaphoreType.DMA,
            pltpu.SemaphoreType.DMA,
            pltpu.SemaphoreType.DMA,
            pltpu.SemaphoreType.DMA,
            pltpu.SemaphoreType.DMA,
            pltpu.SemaphoreType.DMA,
            pltpu.SemaphoreType.DMA,
        ],
    )
    def kern(feats_ref, ids_ref, pidx_ref, starts_ref, out_ref,
             acc_v, rows_v, ids_v, pidx_v, starts_v,
             semg0, semg1, semg2, semg3, si0, si1, si2, si3):
        semg = (semg0, semg1, semg2, semg3)
        si = (si0, si1, si2, si3)
        zvec = jnp.zeros((16,), jnp.float32)
        negvec = jnp.full((16,), NEG, jnp.float32)
        onevec = jnp.ones((16,), jnp.float32)

        pltpu.sync_copy(starts_ref, starts_v)
        wid = lax.axis_index("s") * 2 + lax.axis_index("c")
        c_lo = wid * SEG_W
        wbounds = starts_v[pl.ds(wid, 16)]
        lo = wbounds[0]
        hi = wbounds[1]

        def init(i, _):
            base = i * 128
            for j in range(NCHUNK):
                acc_v[pl.ds(base + j * 16, 16)] = zvec
            return 0

        lax.fori_loop(0, SEG_W + 1, init, 0)

        base_al = (lo // 8) * 8
        nb = (hi - base_al + B - 1) // B
        nb4 = jnp.maximum((nb + 3) // 4, 1)
        nbe = nb4 * 4

        def idx_copies(k, slot):
            base2 = base_al + k * B
            return (
                pltpu.make_async_copy(
                    pidx_ref.at[pl.ds(base2, B)], pidx_v.at[slot], si[slot]),
                pltpu.make_async_copy(
                    ids_ref.at[pl.ds(base2, B)],
                    ids_v.at[pl.ds(slot * B, B)], si[slot]),
            )

        def gather(slot):
            return pltpu.make_async_copy(
                feats_ref.at[pidx_v.at[slot]],
                rows_v.at[pl.ds(slot * B, B)], semg[slot])

        def flush(s, accs, cnt):
            base = s * 128
            for j in range(NCHUNK):
                v = accs[j]
                if is_max:
                    v = lax.select(v == negvec, zvec, v)
                else:
                    v = v / cnt
                acc_v[pl.ds(base + j * 16, 16)] = v

        def process(kb, slot, carry):
            base2 = base_al + kb * B
            rlow = lo - base2
            rhigh = hi - base2

            def gbody(g, carry):
                idvec = ids_v[pl.ds(slot * B + g * 16, 16)]
                if is_max:
                    cur = carry[0]
                    cnt = onevec
                    accs = list(carry[1:])
                else:
                    cur = carry[0]
                    cnt = carry[1]
                    accs = list(carry[2:])
                for l in range(16):
                    r = g * 16 + l
                    valid = (r >= rlow) & (r < rhigh)
                    s_new = jnp.where(valid, idvec[l] - c_lo, SEG_W)
                    boundary = s_new != cur

                    @pl.when(boundary)
                    def _(cur=cur, accs=tuple(accs), cnt=cnt):
                        flush(cur, accs, cnt)

                    rows = []
                    for jj in range(CW // 16):
                        w = rows_v[slot * B + r, pl.ds(jj * 16, 16)]
                        pair = plsc.bitcast(w, jnp.bfloat16)
                        a, bvals = plsc.unpack(
                            pair, format=plsc.PackFormat.INTERLEAVED)
                        rows.append(a)
                        rows.append(bvals)
                    if is_max:
                        bvec = lax.broadcast(
                            jnp.where(boundary, NEG, 0.0).astype(jnp.float32),
                            (16,))
                        accs = [jnp.maximum(accs[j] + bvec, rows[j])
                                for j in range(NCHUNK)]
                    else:
                        mvec = lax.broadcast(
                            jnp.where(boundary, 0.0, 1.0).astype(jnp.float32),
                            (16,))
                        accs = [accs[j] * mvec + rows[j]
                                for j in range(NCHUNK)]
                        cnt = cnt * mvec + onevec
                    cur = s_new
                if is_max:
                    return (cur, *accs)
                return (cur, cnt, *accs)

            return lax.fori_loop(0, B // 16, gbody, carry)

        for s in range(4):
            for cp in idx_copies(jnp.int32(s), s):
                cp.start()
        for s in range(3):
            for cp in idx_copies(jnp.int32(s), s):
                cp.wait()
            gather(s).start()

        if is_max:
            carry0 = (jnp.int32(SEG_W),) + (negvec,) * NCHUNK
        else:
            carry0 = (jnp.int32(SEG_W), onevec) + (zvec,) * NCHUNK

        def body(k4, carry):
            k = k4 * 4
            for b in range(4):
                kb = k + b
                gather(b).wait()
                carry = process(kb, b, carry)

                @pl.when(kb + 4 < nbe)
                def _(kb=kb, b=b):
                    for cp in idx_copies(kb + 4, b):
                        cp.start()

                @pl.when(kb + 3 < nbe)
                def _(kb=kb, b=b):
                    for cp in idx_copies(kb + 3, (b + 3) % 4):
                        cp.wait()
                    gather((b + 3) % 4).start()
            return carry

        carry = lax.fori_loop(0, nb4, body, carry0)

        if is_max:
            flush(carry[0], list(carry[1:]), onevec)
        else:
            flush(carry[0], list(carry[2:]), carry[1])

        pltpu.sync_copy(acc_v.at[pl.ds(0, SEG_W * C)],
                        out_ref.at[pl.ds(c_lo * C, SEG_W * C)])

    return kern


_seg_max = _make_seg_kernel(True)
_seg_mean = _make_seg_kernel(False)


def kernel(feats, cluster_ids, point_idxs, mode):
    bounds = jnp.arange(33, dtype=jnp.int32) * SEG_W
    starts = jnp.searchsorted(cluster_ids, bounds, side="left").astype(jnp.int32)
    starts = jnp.concatenate(
        [starts, jnp.full((15,), SUM_NPOINT, jnp.int32)])
    ids_p = jnp.concatenate(
        [cluster_ids, jnp.zeros((PAD_PTS,), cluster_ids.dtype)])
    pidx_p = jnp.concatenate(
        [point_idxs, jnp.zeros((PAD_PTS,), point_idxs.dtype)])
    feats_packed = lax.bitcast_convert_type(
        feats.astype(jnp.bfloat16).reshape(N_POINTS, CW, 2), jnp.int32)
    args = (feats_packed, ids_p, pidx_p, starts)
    out = lax.cond(mode == 0,
                   lambda: _seg_max(*args),
                   lambda: _seg_mean(*args))
    return out.reshape(OUT_PAD, C)[:N_CLUSTERS, _COL_POS]


# speedup vs baseline: 3.5021x; 3.5021x over previous
import functools

import jax
import jax.numpy as jnp
from jax import lax
from jax.experimental import pallas as pl
from jax.experimental.pallas import tpu as pltpu
from jax.experimental.pallas import tpu_sc as plsc

N_POINTS = 50000
SUM_NPOINT = 320000
C = 128
N_CLUSTERS = 10000

NW = 32
SEG_W = 320
OUT_PAD = NW * SEG_W
B = 128
CW = C // 2
NCHUNK = C // 16
NEG = float("-inf")
PAD_PTS = 640

_CHUNK_POS = [0, 4, 1, 5, 2, 6, 3, 7]

_mesh = plsc.VectorSubcoreMesh(core_axis_name="c", subcore_axis_name="s")


def _make_seg_kernel(is_max):
    @functools.partial(
        pl.kernel,
        mesh=_mesh,
        out_type=jax.ShapeDtypeStruct((OUT_PAD * C,), jnp.float32),
        compiler_params=pltpu.CompilerParams(
            needs_layout_passes=False, use_tc_tiling_on_sc=False),
        scratch_types=[
            pltpu.VMEM(((SEG_W + 1) * C,), jnp.float32),
            pltpu.VMEM((4 * B, CW), jnp.int32),
            pltpu.VMEM((4 * B,), jnp.int32),
            pltpu.VMEM((4, B), jnp.int32),
            pltpu.VMEM((48,), jnp.int32),
            pltpu.SemaphoreType.DMA,
            pltpu.SemaphoreType.DMA,
            pltpu.SemaphoreType.DMA,
            pltpu.SemaphoreType.DMA,
            pltpu.SemaphoreType.DMA,
            pltpu.SemaphoreType.DMA,
            pltpu.SemaphoreType.DMA,
            pltpu.SemaphoreType.DMA,
        ],
    )
    def kern(feats_ref, ids_ref, pidx_ref, starts_ref, out_ref,
             acc_v, rows_v, ids_v, pidx_v, starts_v,
             semg0, semg1, semg2, semg3, si0, si1, si2, si3):
        semg = (semg0, semg1, semg2, semg3)
        si = (si0, si1, si2, si3)
        zvec = jnp.zeros((16,), jnp.float32)
        negvec = jnp.full((16,), NEG, jnp.float32)
        onevec = jnp.ones((16,), jnp.float32)

        pltpu.sync_copy(starts_ref, starts_v)
        wid = lax.axis_index("s") * 2 + lax.axis_index("c")
        c_lo = wid * SEG_W
        wbounds = starts_v[pl.ds(wid, 16)]
        lo = wbounds[0]
        hi = wbounds[1]

        def init(i, _):
            base = i * 128
            for j in range(NCHUNK):
                acc_v[pl.ds(base + j * 16, 16)] = zvec
            return 0

        lax.fori_loop(0, SEG_W + 1, init, 0)

        base_al = (lo // 8) * 8
        nb = (hi - base_al + B - 1) // B
        nb4 = jnp.maximum((nb + 3) // 4, 1)
        nbe = nb4 * 4

        def idx_copies(k, slot):
            base2 = base_al + k * B
            return (
                pltpu.make_async_copy(
                    pidx_ref.at[pl.ds(base2, B)], pidx_v.at[slot], si[slot]),
                pltpu.make_async_copy(
                    ids_ref.at[pl.ds(base2, B)],
                    ids_v.at[pl.ds(slot * B, B)], si[slot]),
            )

        def gather(slot):
            return pltpu.make_async_copy(
                feats_ref.at[pidx_v.at[slot]],
                rows_v.at[pl.ds(slot * B, B)], semg[slot])

        def flush(s, accs, cnt):
            base = s * 128
            for j in range(NCHUNK):
                v = accs[j]
                if is_max:
                    v = lax.select(v == negvec, zvec, v)
                else:
                    v = v / cnt
                acc_v[pl.ds(base + _CHUNK_POS[j] * 16, 16)] = v

        def process(kb, slot, carry):
            base2 = base_al + kb * B
            rlow = lo - base2
            rhigh = hi - base2

            def gbody(g, carry):
                idvec = ids_v[pl.ds(slot * B + g * 16, 16)]
                if is_max:
                    cur = carry[0]
                    cnt = onevec
                    accs = list(carry[1:])
                else:
                    cur = carry[0]
                    cnt = carry[1]
                    accs = list(carry[2:])
                for l in range(16):
                    r = g * 16 + l
                    valid = (r >= rlow) & (r < rhigh)
                    s_new = jnp.where(valid, idvec[l] - c_lo, SEG_W)
                    boundary = s_new != cur

                    @pl.when(boundary)
                    def _(cur=cur, accs=tuple(accs), cnt=cnt):
                        flush(cur, accs, cnt)

                    rows = []
                    for jj in range(CW // 16):
                        w = rows_v[slot * B + r, pl.ds(jj * 16, 16)]
                        pair = plsc.bitcast(w, jnp.bfloat16)
                        a, bvals = plsc.unpack(
                            pair, format=plsc.PackFormat.INTERLEAVED)
                        rows.append(a)
                        rows.append(bvals)
                    if is_max:
                        bvec = lax.broadcast(
                            jnp.where(boundary, NEG, 0.0).astype(jnp.float32),
                            (16,))
                        accs = [jnp.maximum(accs[j] + bvec, rows[j])
                                for j in range(NCHUNK)]
                    else:
                        mvec = lax.broadcast(
                            jnp.where(boundary, 0.0, 1.0).astype(jnp.float32),
                            (16,))
                        accs = [accs[j] * mvec + rows[j]
                                for j in range(NCHUNK)]
                        cnt = cnt * mvec + onevec
                    cur = s_new
                if is_max:
                    return (cur, *accs)
                return (cur, cnt, *accs)

            return lax.fori_loop(0, B // 16, gbody, carry)

        for s in range(4):
            for cp in idx_copies(jnp.int32(s), s):
                cp.start()
        for s in range(3):
            for cp in idx_copies(jnp.int32(s), s):
                cp.wait()
            gather(s).start()

        if is_max:
            carry0 = (jnp.int32(SEG_W),) + (negvec,) * NCHUNK
        else:
            carry0 = (jnp.int32(SEG_W), onevec) + (zvec,) * NCHUNK

        def body(k4, carry):
            k = k4 * 4
            for b in range(4):
                kb = k + b
                gather(b).wait()
                carry = process(kb, b, carry)

                @pl.when(kb + 4 < nbe)
                def _(kb=kb, b=b):
                    for cp in idx_copies(kb + 4, b):
                        cp.start()

                @pl.when(kb + 3 < nbe)
                def _(kb=kb, b=b):
                    for cp in idx_copies(kb + 3, (b + 3) % 4):
                        cp.wait()
                    gather((b + 3) % 4).start()
            return carry

        carry = lax.fori_loop(0, nb4, body, carry0)

        if is_max:
            flush(carry[0], list(carry[1:]), onevec)
        else:
            flush(carry[0], list(carry[2:]), carry[1])

        pltpu.sync_copy(acc_v.at[pl.ds(0, SEG_W * C)],
                        out_ref.at[pl.ds(c_lo * C, SEG_W * C)])

    return kern


_seg_max = _make_seg_kernel(True)
_seg_mean = _make_seg_kernel(False)


def kernel(feats, cluster_ids, point_idxs, mode):
    bounds = jnp.arange(33, dtype=jnp.int32) * SEG_W
    starts = jnp.searchsorted(cluster_ids, bounds, side="left").astype(jnp.int32)
    starts = jnp.concatenate(
        [starts, jnp.full((15,), SUM_NPOINT, jnp.int32)])
    ids_p = jnp.concatenate(
        [cluster_ids, jnp.zeros((PAD_PTS,), cluster_ids.dtype)])
    pidx_p = jnp.concatenate(
        [point_idxs, jnp.zeros((PAD_PTS,), point_idxs.dtype)])
    u = lax.bitcast_convert_type(feats, jnp.uint32)

    def _rnd(x):
        return (x + jnp.uint32(0x7FFF) + ((x >> jnp.uint32(16))
                                          & jnp.uint32(1))) >> jnp.uint32(16)

    w = (_rnd(u[:, CW:]) << jnp.uint32(16)) | _rnd(u[:, :CW])
    feats_packed = lax.bitcast_convert_type(w, jnp.int32)
    args = (feats_packed, ids_p, pidx_p, starts)
    out = lax.cond(mode == 0,
                   lambda: _seg_max(*args),
                   lambda: _seg_mean(*args))
    return out.reshape(OUT_PAD, C)[:N_CLUSTERS]


# speedup vs baseline: 4.9267x; 1.4068x over previous
import functools

import jax
import jax.numpy as jnp
from jax import lax
from jax.experimental import pallas as pl
from jax.experimental.pallas import tpu as pltpu
from jax.experimental.pallas import tpu_sc as plsc

N_POINTS = 50000
SUM_NPOINT = 320000
C = 128
N_CLUSTERS = 10000

NW = 32
SEG_W = 320
OUT_PAD = NW * SEG_W
B = 128
NCHUNK = C // 16
NEG = float("-inf")
PAD_PTS = 640

_mesh = plsc.VectorSubcoreMesh(core_axis_name="c", subcore_axis_name="s")


def _make_seg_kernel(is_max):
    @functools.partial(
        pl.kernel,
        mesh=_mesh,
        out_type=jax.ShapeDtypeStruct((OUT_PAD * C,), jnp.float32),
        compiler_params=pltpu.CompilerParams(
            needs_layout_passes=False, use_tc_tiling_on_sc=False),
        scratch_types=[
            pltpu.VMEM(((SEG_W + 1) * C,), jnp.float32),
            pltpu.VMEM((4 * B, C), jnp.float32),
            pltpu.VMEM((4 * B,), jnp.int32),
            pltpu.VMEM((4, B), jnp.int32),
            pltpu.VMEM((48,), jnp.int32),
            pltpu.SemaphoreType.DMA,
            pltpu.SemaphoreType.DMA,
            pltpu.SemaphoreType.DMA,
            pltpu.SemaphoreType.DMA,
            pltpu.SemaphoreType.DMA,
            pltpu.SemaphoreType.DMA,
            pltpu.SemaphoreType.DMA,
            pltpu.SemaphoreType.DMA,
        ],
    )
    def kern(feats_ref, ids_ref, pidx_ref, starts_ref, out_ref,
             acc_v, rows_v, ids_v, pidx_v, starts_v,
             semg0, semg1, semg2, semg3, si0, si1, si2, si3):
        semg = (semg0, semg1, semg2, semg3)
        si = (si0, si1, si2, si3)
        zvec = jnp.zeros((16,), jnp.float32)
        negvec = jnp.full((16,), NEG, jnp.float32)
        onevec = jnp.ones((16,), jnp.float32)

        pltpu.sync_copy(starts_ref, starts_v)
        wid = lax.axis_index("s") * 2 + lax.axis_index("c")
        c_lo = wid * SEG_W
        wbounds = starts_v[pl.ds(wid, 16)]
        lo = wbounds[0]
        hi = wbounds[1]

        def init(i, _):
            base = i * 128
            for j in range(NCHUNK):
                acc_v[pl.ds(base + j * 16, 16)] = zvec
            return 0

        lax.fori_loop(0, SEG_W + 1, init, 0)

        base_al = (lo // 8) * 8
        nb = (hi - base_al + B - 1) // B
        nb4 = jnp.maximum((nb + 3) // 4, 1)
        nbe = nb4 * 4

        def idx_copies(k, slot):
            base2 = base_al + k * B
            return (
                pltpu.make_async_copy(
                    pidx_ref.at[pl.ds(base2, B)], pidx_v.at[slot], si[slot]),
                pltpu.make_async_copy(
                    ids_ref.at[pl.ds(base2, B)],
                    ids_v.at[pl.ds(slot * B, B)], si[slot]),
            )

        def gather(slot):
            return pltpu.make_async_copy(
                feats_ref.at[pidx_v.at[slot]],
                rows_v.at[pl.ds(slot * B, B)], semg[slot])

        def flush(s, accs, cnt):
            base = s * 128
            for j in range(NCHUNK):
                v = accs[j]
                if is_max:
                    v = lax.select(v == negvec, zvec, v)
                else:
                    v = v / cnt
                acc_v[pl.ds(base + j * 16, 16)] = v

        def process(kb, slot, carry):
            base2 = base_al + kb * B
            rlow = lo - base2
            rhigh = hi - base2

            def gbody(g, carry):
                idvec = ids_v[pl.ds(slot * B + g * 16, 16)]
                if is_max:
                    cur = carry[0]
                    cnt = onevec
                    accs = list(carry[1:])
                else:
                    cur = carry[0]
                    cnt = carry[1]
                    accs = list(carry[2:])
                for l in range(16):
                    r = g * 16 + l
                    valid = (r >= rlow) & (r < rhigh)
                    s_new = jnp.where(valid, idvec[l] - c_lo, SEG_W)
                    boundary = s_new != cur

                    @pl.when(boundary)
                    def _(cur=cur, accs=tuple(accs), cnt=cnt):
                        flush(cur, accs, cnt)

                    rows = [rows_v[slot * B + r, pl.ds(j * 16, 16)]
                            for j in range(NCHUNK)]
                    if is_max:
                        bvec = lax.broadcast(
                            jnp.where(boundary, NEG, 0.0).astype(jnp.float32),
                            (16,))
                        accs = [jnp.maximum(accs[j] + bvec, rows[j])
                                for j in range(NCHUNK)]
                    else:
                        mvec = lax.broadcast(
                            jnp.where(boundary, 0.0, 1.0).astype(jnp.float32),
                            (16,))
                        accs = [accs[j] * mvec + rows[j]
                                for j in range(NCHUNK)]
                        cnt = cnt * mvec + onevec
                    cur = s_new
                if is_max:
                    return (cur, *accs)
                return (cur, cnt, *accs)

            return lax.fori_loop(0, B // 16, gbody, carry)

        for s in range(4):
            for cp in idx_copies(jnp.int32(s), s):
                cp.start()
        for s in range(3):
            for cp in idx_copies(jnp.int32(s), s):
                cp.wait()
            gather(s).start()

        if is_max:
            carry0 = (jnp.int32(SEG_W),) + (negvec,) * NCHUNK
        else:
            carry0 = (jnp.int32(SEG_W), onevec) + (zvec,) * NCHUNK

        def body(k4, carry):
            k = k4 * 4
            for b in range(4):
                kb = k + b
                gather(b).wait()
                carry = process(kb, b, carry)

                @pl.when(kb + 4 < nbe)
                def _(kb=kb, b=b):
                    for cp in idx_copies(kb + 4, b):
                        cp.start()

                @pl.when(kb + 3 < nbe)
                def _(kb=kb, b=b):
                    for cp in idx_copies(kb + 3, (b + 3) % 4):
                        cp.wait()
                    gather((b + 3) % 4).start()
            return carry

        carry = lax.fori_loop(0, nb4, body, carry0)

        if is_max:
            flush(carry[0], list(carry[1:]), onevec)
        else:
            flush(carry[0], list(carry[2:]), carry[1])

        pltpu.sync_copy(acc_v.at[pl.ds(0, SEG_W * C)],
                        out_ref.at[pl.ds(c_lo * C, SEG_W * C)])

    return kern


_seg_max = _make_seg_kernel(True)
_seg_mean = _make_seg_kernel(False)


def kernel(feats, cluster_ids, point_idxs, mode):
    bounds = jnp.arange(33, dtype=jnp.int32) * SEG_W
    starts = jnp.searchsorted(cluster_ids, bounds, side="left").astype(jnp.int32)
    starts = jnp.concatenate(
        [starts, jnp.full((15,), SUM_NPOINT, jnp.int32)])
    ids_p = jnp.concatenate(
        [cluster_ids, jnp.zeros((PAD_PTS,), cluster_ids.dtype)])
    pidx_p = jnp.concatenate(
        [point_idxs, jnp.zeros((PAD_PTS,), point_idxs.dtype)])
    args = (feats, ids_p, pidx_p, starts)
    out = lax.cond(mode == 0,
                   lambda: _seg_max(*args),
                   lambda: _seg_mean(*args))
    return out.reshape(OUT_PAD, C)[:N_CLUSTERS]
